# Initial kernel scaffold; baseline (speedup 1.0000x reference)
import jax
import jax.numpy as jnp
from jax.experimental import pallas as pl


def kernel(idx, targets, table):
    raise NotImplementedError("write your pallas kernel here")


# speedup vs baseline: 1.7854x; 1.7854x over previous
import functools

import jax
import jax.numpy as jnp
from jax import lax
from jax.experimental import pallas as pl

VOCAB = 1000
N_TOK = 1024 * 50
BLK = 512
NBLK = N_TOK // BLK


def _tc_body(idx_ref, tgt_ref, table_ref, out_ref, loss_ref):
    i = pl.program_id(0)
    idxv = idx_ref[0]
    tgtv = tgt_ref[0]
    cols = lax.broadcasted_iota(jnp.int32, (BLK, VOCAB), 1)
    onehot = (cols == idxv).astype(jnp.float32)
    logits = jnp.dot(onehot, table_ref[...], preferred_element_type=jnp.float32)
    out_ref[...] = logits
    m = jnp.max(logits, axis=1, keepdims=True)
    s = jnp.sum(jnp.exp(logits - m), axis=1, keepdims=True)
    lse = m + jnp.log(s)
    tlogit = jnp.sum(jnp.where(cols == tgtv, logits, 0.0), axis=1, keepdims=True)
    nll_sum = jnp.sum(lse - tlogit).reshape(1, 1)

    @pl.when(i == 0)
    def _():
        loss_ref[...] = jnp.zeros((1, 1), jnp.float32)

    loss_ref[...] += nll_sum


@jax.jit
def kernel(idx, targets, table):
    idx_r = idx.reshape(NBLK, BLK, 1)
    tgt_r = targets.reshape(NBLK, BLK, 1)
    logits_flat, loss_sum = pl.pallas_call(
        _tc_body,
        grid=(NBLK,),
        in_specs=[
            pl.BlockSpec((1, BLK, 1), lambda i: (i, 0, 0)),
            pl.BlockSpec((1, BLK, 1), lambda i: (i, 0, 0)),
            pl.BlockSpec((VOCAB, VOCAB), lambda i: (0, 0)),
        ],
        out_specs=[
            pl.BlockSpec((BLK, VOCAB), lambda i: (i, 0)),
            pl.BlockSpec((1, 1), lambda i: (0, 0)),
        ],
        out_shape=[
            jax.ShapeDtypeStruct((N_TOK, VOCAB), jnp.float32),
            jax.ShapeDtypeStruct((1, 1), jnp.float32),
        ],
    )(idx_r, tgt_r, table)
    logits = logits_flat.reshape(idx.shape[0], idx.shape[1], VOCAB)
    loss = loss_sum[0, 0] / N_TOK
    return (logits, loss)
